# baseline (speedup 1.0000x reference)
import functools

import jax
import jax.numpy as jnp
from jax import lax
from jax.experimental import pallas as pl
from jax.experimental.pallas import tpu as pltpu
from jax.experimental.pallas import tpu_sc as plsc

GRID = 64
N = GRID ** 3
C = 8
H = W = 512
HW = H * W
NW = 32
PPW = N // NW
SLICES = 4
SLICE_PIX = HW // SLICES
FILL = 0.45
NONVIS_W = 0.25
HALF = (GRID ** 3) // 32 // 2
MAGIC = 12582912.0
F32_INF = float("inf")

_CPARAMS = pltpu.CompilerParams(needs_layout_passes=False)
_MESH = plsc.VectorSubcoreMesh(core_axis_name="c", subcore_axis_name="s")


def _wid():
    return lax.axis_index("c") * 16 + lax.axis_index("s")


def _any(v_bool):
    return jnp.max(v_bool.astype(jnp.int32), axis=0) != 0


NPAR = 24


def _prep_body(bx_h, by_h, bz_h, par_h, glob_h, pid_h, dist_h,
               bxv, byv, bzv, pidv, dstv, parv, globv, sem):
    w = _wid()
    base = w * PPW
    pltpu.sync_copy(bx_h.at[pl.ds(base, PPW)], bxv)
    pltpu.sync_copy(by_h.at[pl.ds(base, PPW)], byv)
    pltpu.sync_copy(bz_h.at[pl.ds(base, PPW)], bzv)
    pltpu.sync_copy(par_h, parv)
    pltpu.sync_copy(glob_h, globv)

    ct = globv[pl.ds(0, 16)]
    st = globv[pl.ds(16, 16)]
    cx = globv[pl.ds(32, 16)]
    cy = globv[pl.ds(48, 16)]
    cz = globv[pl.ds(64, 16)]

    for c in range(C):
        def P(j, c=c):
            return parv[pl.ds((c * NPAR + j) * 16, 16)]

        e00 = P(0)
        e01 = P(1)
        e02 = P(2)
        e03 = P(3)
        e10 = P(4)
        e11 = P(5)
        e12 = P(6)
        e13 = P(7)
        e20 = P(8)
        e21 = P(9)
        e22 = P(10)
        e23 = P(11)
        k00 = P(12)
        k01 = P(13)
        k02 = P(14)
        k10 = P(15)
        k11 = P(16)
        k12 = P(17)
        k20 = P(18)
        k21 = P(19)
        k22 = P(20)
        px0 = P(21)
        py0 = P(22)
        pz0 = P(23)

        def body(i, _, c=c):
            sl = pl.ds(i * 16, 16)
            bxx = bxv[sl]
            byy = byv[sl]
            bzz = bzv[sl]
            x = ct * bxx - st * byy + cx
            y = st * bxx + ct * byy + cy
            z = bzz + cz
            camx = e00 * x + e01 * y + e02 * z + e03
            camy = e10 * x + e11 * y + e12 * z + e13
            camz = e20 * x + e21 * y + e22 * z + e23
            pixx = k00 * camx + k01 * camy + k02 * camz
            pixy = k10 * camx + k11 * camy + k12 * camz
            pixz = k20 * camx + k21 * camy + k22 * camz
            zc = pixz + 1e-8
            u = pixx / zc
            v = pixy / zc
            ru = (u + MAGIC) - MAGIC
            rv = (v + MAGIC) - MAGIC
            ru = jnp.minimum(jnp.maximum(ru, 0.0), 511.0)
            rv = jnp.minimum(jnp.maximum(rv, 0.0), 511.0)
            ipx = ru.astype(jnp.int32)
            ipy = rv.astype(jnp.int32)
            pidv[sl] = ipy * W + ipx
            dx = x - px0
            dy = y - py0
            dz = z - pz0
            dstv[sl] = dx * dx + dy * dy + dz * dz
            return 0

        lax.fori_loop(0, PPW // 16, body, 0)
        pltpu.sync_copy(pidv, pid_h.at[c, pl.ds(base, PPW)])
        pltpu.sync_copy(dstv, dist_h.at[c, pl.ds(base, PPW)])


def _zbuf_body(pid_h, dist_h, zbuf_h, pidc, dstc, zs, sem):
    w = _wid()
    cam = w // SLICES
    sl_i = w % SLICES
    pid_base = sl_i * SLICE_PIX

    def init(i, _):
        zs[pl.ds(i * 16, 16)] = jnp.full((16,), F32_INF, jnp.float32)
        return 0

    lax.fori_loop(0, SLICE_PIX // 16, init, 0)

    CH = 16384
    for chunk in range(N // CH):
        pltpu.sync_copy(pid_h.at[cam, pl.ds(chunk * CH, CH)], pidc)
        pltpu.sync_copy(dist_h.at[cam, pl.ds(chunk * CH, CH)], dstc)

        def body(i, _):
            sl = pl.ds(i * 16, 16)
            pv = pidc[sl]
            dv = dstc[sl]
            li = pv - pid_base
            m = (li >= 0) & (li < SLICE_PIX)
            dvm = jnp.where(m, dv, F32_INF)
            lic = jnp.minimum(jnp.maximum(li, 0), SLICE_PIX - 1)
            ds_, ls_ = plsc.sort_key_val(dvm, lic, descending=True)
            cur = plsc.load_gather(zs, [ls_])
            wm = ds_ < cur
            plsc.store_scatter(zs, [ls_], ds_, mask=wm)
            return 0

        lax.fori_loop(0, CH // 16, body, 0)

    pltpu.sync_copy(zs, zbuf_h.at[cam, pl.ds(pid_base, SLICE_PIX)])


def _sample_body(pid_h, dist_h, zbuf_h, img4_h, out_h,
                 pidb, dstb, idxzh, idx4, zgh, pix4,
                 msum, wsum, cr, cg, cb, sem, sem2):
    w = _wid()
    base = w * PPW

    def zero(i, _):
        sl = pl.ds(i * 16, 16)
        zv = jnp.zeros((16,), jnp.float32)
        msum[sl] = zv
        wsum[sl] = zv
        cr[sl] = zv
        cg[sl] = zv
        cb[sl] = zv
        return 0

    lax.fori_loop(0, PPW // 16, zero, 0)

    for c in range(C):
        pltpu.sync_copy(pid_h.at[c, pl.ds(base, PPW)], pidb)
        pltpu.sync_copy(dist_h.at[c, pl.ds(base, PPW)], dstb)

        for h in range(2):
            def mk(i, _, c=c, h=h):
                pts4 = (lax.iota(jnp.int32, 16) + (i * 16)) * 4
                pv = pidb[pl.ds(h * HALF + i * 16, 16)] + (c * HW)
                idxzh[pl.ds(i * 16, 16)] = pv
                pv4 = pv * 4
                plsc.store_scatter(idx4, [pts4], pv4)
                plsc.store_scatter(idx4, [pts4 + 1], pv4 + 1)
                plsc.store_scatter(idx4, [pts4 + 2], pv4 + 2)
                plsc.store_scatter(idx4, [pts4 + 3], pv4 + 3)
                return 0

            lax.fori_loop(0, HALF // 16, mk, 0)

            d1 = pltpu.async_copy(zbuf_h.at[idxzh], zgh, sem)
            d2 = pltpu.async_copy(img4_h.at[idx4], pix4, sem2)
            d1.wait()
            d2.wait()

            def acc(i, _, h=h):
                slg = pl.ds(h * HALF + i * 16, 16)
                sll = pl.ds(i * 16, 16)
                pts4 = (lax.iota(jnp.int32, 16) + (i * 16)) * 4
                mgv = plsc.load_gather(pix4, [pts4])
                rgv = plsc.load_gather(pix4, [pts4 + 1])
                ggv = plsc.load_gather(pix4, [pts4 + 2])
                bgv = plsc.load_gather(pix4, [pts4 + 3])
                vis = dstb[slg] <= zgh[sll]
                wv = jnp.where(vis, 1.0, NONVIS_W).astype(jnp.float32)
                msum[slg] = msum[slg] + mgv
                wsum[slg] = wsum[slg] + wv
                cr[slg] = cr[slg] + wv * rgv
                cg[slg] = cg[slg] + wv * ggv
                cb[slg] = cb[slg] + wv * bgv
                return 0

            lax.fori_loop(0, HALF // 16, acc, 0)

    def fin(i, _):
        sl = pl.ds(i * 16, 16)
        mv = msum[sl] * (1.0 / C)
        b1 = mv >= 1.0
        b2 = mv >= (C - 1) / C
        den = jnp.maximum(wsum[sl], 1e-8)
        colr = cr[sl] / den
        colg = cg[sl] / den
        colb = cb[sl] / den
        msum[sl] = b1.astype(jnp.float32) * 0.5 + b2.astype(jnp.float32) * 0.5
        wsum[sl] = jnp.where(b1, colr, FILL) * 0.5 + jnp.where(b2, colr, FILL) * 0.5
        cr[sl] = jnp.where(b1, colg, FILL) * 0.5 + jnp.where(b2, colg, FILL) * 0.5
        cg[sl] = jnp.where(b1, colb, FILL) * 0.5 + jnp.where(b2, colb, FILL) * 0.5
        return 0

    lax.fori_loop(0, PPW // 16, fin, 0)
    pltpu.sync_copy(msum, out_h.at[0, pl.ds(base, PPW)])
    pltpu.sync_copy(wsum, out_h.at[1, pl.ds(base, PPW)])
    pltpu.sync_copy(cr, out_h.at[2, pl.ds(base, PPW)])
    pltpu.sync_copy(cg, out_h.at[3, pl.ds(base, PPW)])


@functools.partial(jax.jit, static_argnames=())
def _run(img4, bx, by, bz, par, glob):
    prep = pl.kernel(
        _prep_body,
        mesh=_MESH,
        compiler_params=_CPARAMS,
        out_type=(
            jax.ShapeDtypeStruct((C, N), jnp.int32),
            jax.ShapeDtypeStruct((C, N), jnp.float32),
        ),
        scratch_types=[
            pltpu.VMEM((PPW,), jnp.float32),
            pltpu.VMEM((PPW,), jnp.float32),
            pltpu.VMEM((PPW,), jnp.float32),
            pltpu.VMEM((PPW,), jnp.int32),
            pltpu.VMEM((PPW,), jnp.float32),
            pltpu.VMEM((C * NPAR * 16,), jnp.float32),
            pltpu.VMEM((5 * 16,), jnp.float32),
            pltpu.SemaphoreType.DMA,
        ],
    )
    pid, dist = prep(bx, by, bz, par, glob)

    zbuf = pl.kernel(
        _zbuf_body,
        mesh=_MESH,
        compiler_params=_CPARAMS,
        out_type=jax.ShapeDtypeStruct((C, HW), jnp.float32),
        scratch_types=[
            pltpu.VMEM((16384,), jnp.int32),
            pltpu.VMEM((16384,), jnp.float32),
            pltpu.VMEM((SLICE_PIX,), jnp.float32),
            pltpu.SemaphoreType.DMA,
        ],
    )(pid, dist)


    out4 = pl.kernel(
        _sample_body,
        mesh=_MESH,
        compiler_params=_CPARAMS,
        out_type=jax.ShapeDtypeStruct((4, N), jnp.float32),
        scratch_types=(
            [pltpu.VMEM((PPW,), jnp.int32),
             pltpu.VMEM((PPW,), jnp.float32),
             pltpu.VMEM((HALF,), jnp.int32),
             pltpu.VMEM((4 * HALF,), jnp.int32),
             pltpu.VMEM((HALF,), jnp.float32),
             pltpu.VMEM((4 * HALF,), jnp.float32)]
            + [pltpu.VMEM((PPW,), jnp.float32)] * 5
            + [pltpu.SemaphoreType.DMA, pltpu.SemaphoreType.DMA]
        ),
    )(pid, dist, zbuf.reshape(C * HW), img4)
    return out4


def kernel(mask, rgb, center, angle, K, E):
    lin = jnp.linspace(-0.5, 0.5, GRID, dtype=jnp.float32)
    gx, gy, gz = jnp.meshgrid(lin, lin, lin, indexing="ij")
    bx = gx.reshape(-1)
    by = gy.reshape(-1)
    bz = gz.reshape(-1)

    th = jnp.asarray(angle, jnp.float32)
    ct, st = jnp.cos(th), jnp.sin(th)
    Rt = jnp.transpose(E[:, :3, :3], (0, 2, 1))
    cam_pos = -jnp.einsum("cij,cj->ci", Rt, E[:, :3, 3])

    par = jnp.concatenate(
        [
            E[:, :3, :].reshape(C, 12),
            K.reshape(C, 9),
            cam_pos.reshape(C, 3),
        ],
        axis=1,
    )
    par = jnp.broadcast_to(par[:, :, None], (C, NPAR, 16)).reshape(-1)
    glob = jnp.concatenate(
        [ct[None], st[None], jnp.asarray(center, jnp.float32)]
    )
    glob = jnp.broadcast_to(glob[:, None], (5, 16)).reshape(-1)

    img4 = jnp.stack(
        [mask.reshape(C * HW), rgb[:, 0].reshape(-1), rgb[:, 1].reshape(-1),
         rgb[:, 2].reshape(-1)], axis=-1).reshape(-1)
    out4 = _run(img4, bx, by, bz, par, glob)
    return out4.reshape(4, GRID, GRID, GRID)


# speedup vs baseline: 4.0355x; 4.0355x over previous
import functools

import jax
import jax.numpy as jnp
from jax import lax
from jax.experimental import pallas as pl
from jax.experimental.pallas import tpu as pltpu
from jax.experimental.pallas import tpu_sc as plsc

GRID = 64
N = GRID ** 3
C = 8
H = W = 512
HW = H * W
NW = 32
PPW = N // NW
SLICES = 4
SLICE_PIX = HW // SLICES
FILL = 0.45
NONVIS_W = 0.25
HALF = (GRID ** 3) // 32 // 2
MAGIC = 12582912.0
F32_INF = float("inf")

_CPARAMS = pltpu.CompilerParams(needs_layout_passes=False)
_MESH = plsc.VectorSubcoreMesh(core_axis_name="c", subcore_axis_name="s")


def _wid():
    return lax.axis_index("c") * 16 + lax.axis_index("s")


def _any(v_bool):
    return jnp.max(v_bool.astype(jnp.int32), axis=0) != 0


NPAR = 24


def _prep_body(bx_h, by_h, bz_h, par_h, glob_h, pid_h, dist_h,
               bxv, byv, bzv, pidv, dstv, parv, globv, sem):
    w = _wid()
    base = w * PPW
    pltpu.sync_copy(bx_h.at[pl.ds(base, PPW)], bxv)
    pltpu.sync_copy(by_h.at[pl.ds(base, PPW)], byv)
    pltpu.sync_copy(bz_h.at[pl.ds(base, PPW)], bzv)
    pltpu.sync_copy(par_h, parv)
    pltpu.sync_copy(glob_h, globv)

    ct = globv[pl.ds(0, 16)]
    st = globv[pl.ds(16, 16)]
    cx = globv[pl.ds(32, 16)]
    cy = globv[pl.ds(48, 16)]
    cz = globv[pl.ds(64, 16)]

    for c in range(C):
        def P(j, c=c):
            return parv[pl.ds((c * NPAR + j) * 16, 16)]

        e00 = P(0)
        e01 = P(1)
        e02 = P(2)
        e03 = P(3)
        e10 = P(4)
        e11 = P(5)
        e12 = P(6)
        e13 = P(7)
        e20 = P(8)
        e21 = P(9)
        e22 = P(10)
        e23 = P(11)
        k00 = P(12)
        k01 = P(13)
        k02 = P(14)
        k10 = P(15)
        k11 = P(16)
        k12 = P(17)
        k20 = P(18)
        k21 = P(19)
        k22 = P(20)
        px0 = P(21)
        py0 = P(22)
        pz0 = P(23)

        def body(i, _, c=c):
            sl = pl.ds(i * 16, 16)
            bxx = bxv[sl]
            byy = byv[sl]
            bzz = bzv[sl]
            x = ct * bxx - st * byy + cx
            y = st * bxx + ct * byy + cy
            z = bzz + cz
            camx = e00 * x + e01 * y + e02 * z + e03
            camy = e10 * x + e11 * y + e12 * z + e13
            camz = e20 * x + e21 * y + e22 * z + e23
            pixx = k00 * camx + k01 * camy + k02 * camz
            pixy = k10 * camx + k11 * camy + k12 * camz
            pixz = k20 * camx + k21 * camy + k22 * camz
            zc = pixz + 1e-8
            u = pixx / zc
            v = pixy / zc
            ru = (u + MAGIC) - MAGIC
            rv = (v + MAGIC) - MAGIC
            ru = jnp.minimum(jnp.maximum(ru, 0.0), 511.0)
            rv = jnp.minimum(jnp.maximum(rv, 0.0), 511.0)
            ipx = ru.astype(jnp.int32)
            ipy = rv.astype(jnp.int32)
            pidv[sl] = ipy * W + ipx
            dx = x - px0
            dy = y - py0
            dz = z - pz0
            dstv[sl] = dx * dx + dy * dy + dz * dz
            return 0

        lax.fori_loop(0, PPW // 16, body, 0)
        pltpu.sync_copy(pidv, pid_h.at[c, pl.ds(base, PPW)])
        pltpu.sync_copy(dstv, dist_h.at[c, pl.ds(base, PPW)])


def _zbuf_body(pid_h, dist_h, zbuf_h, pidc, dstc, zs, sem):
    w = _wid()
    cam = w // SLICES
    sl_i = w % SLICES
    pid_base = sl_i * SLICE_PIX

    def init(i, _):
        zs[pl.ds(i * 16, 16)] = jnp.full((16,), F32_INF, jnp.float32)
        return 0

    lax.fori_loop(0, SLICE_PIX // 16, init, 0)

    CH = 16384
    for chunk in range(N // CH):
        pltpu.sync_copy(pid_h.at[cam, pl.ds(chunk * CH, CH)], pidc)
        pltpu.sync_copy(dist_h.at[cam, pl.ds(chunk * CH, CH)], dstc)

        UNROLL = 4

        def body(i, _):
            sorted_vs = []
            for u in range(UNROLL):
                sl = pl.ds(i * (16 * UNROLL) + u * 16, 16)
                pv = pidc[sl]
                dv = dstc[sl]
                li = pv - pid_base
                m = (li >= 0) & (li < SLICE_PIX)
                dvm = jnp.where(m, dv, F32_INF)
                lic = jnp.minimum(jnp.maximum(li, 0), SLICE_PIX - 1)
                sorted_vs.append(plsc.sort_key_val(dvm, lic, descending=True))
            for ds_, ls_ in sorted_vs:
                cur = plsc.load_gather(zs, [ls_])
                wm = ds_ < cur
                plsc.store_scatter(zs, [ls_], ds_, mask=wm)
            return 0

        lax.fori_loop(0, CH // (16 * UNROLL), body, 0)

    pltpu.sync_copy(zs, zbuf_h.at[cam, pl.ds(pid_base, SLICE_PIX)])


def _sample_body(pid_h, dist_h, zbuf_h, mask_h, r_h, g_h, b_h, out_h,
                 pidb, idxz, dstb, zg, mg, rg, gg, bg,
                 msum, wsum, cr, cg, cb, sem, sem2):
    w = _wid()
    base = w * PPW

    def zero(i, _):
        sl = pl.ds(i * 16, 16)
        zv = jnp.zeros((16,), jnp.float32)
        msum[sl] = zv
        wsum[sl] = zv
        cr[sl] = zv
        cg[sl] = zv
        cb[sl] = zv
        return 0

    lax.fori_loop(0, PPW // 16, zero, 0)

    for c in range(C):
        pltpu.sync_copy(pid_h.at[c, pl.ds(base, PPW)], pidb)
        pltpu.sync_copy(dist_h.at[c, pl.ds(base, PPW)], dstb)

        def off(i, _, c=c):
            sl = pl.ds(i * 16, 16)
            idxz[sl] = pidb[sl] + (c * HW)
            return 0

        lax.fori_loop(0, PPW // 16, off, 0)

        d1 = pltpu.async_copy(zbuf_h.at[idxz], zg, sem)
        d2 = pltpu.async_copy(mask_h.at[idxz], mg, sem2)
        d3 = pltpu.async_copy(r_h.at[idxz], rg, sem)
        d4 = pltpu.async_copy(g_h.at[idxz], gg, sem2)
        d5 = pltpu.async_copy(b_h.at[idxz], bg, sem)
        d1.wait()
        d2.wait()
        d3.wait()
        d4.wait()
        d5.wait()

        def acc(i, _):
            sl = pl.ds(i * 16, 16)
            vis = dstb[sl] <= zg[sl]
            wv = jnp.where(vis, 1.0, NONVIS_W).astype(jnp.float32)
            msum[sl] = msum[sl] + mg[sl]
            wsum[sl] = wsum[sl] + wv
            cr[sl] = cr[sl] + wv * rg[sl]
            cg[sl] = cg[sl] + wv * gg[sl]
            cb[sl] = cb[sl] + wv * bg[sl]
            return 0

        lax.fori_loop(0, PPW // 16, acc, 0)

    def fin(i, _):
        sl = pl.ds(i * 16, 16)
        mv = msum[sl] * (1.0 / C)
        b1 = mv >= 1.0
        b2 = mv >= (C - 1) / C
        den = jnp.maximum(wsum[sl], 1e-8)
        colr = cr[sl] / den
        colg = cg[sl] / den
        colb = cb[sl] / den
        zg[sl] = b1.astype(jnp.float32) * 0.5 + b2.astype(jnp.float32) * 0.5
        mg[sl] = jnp.where(b1, colr, FILL) * 0.5 + jnp.where(b2, colr, FILL) * 0.5
        rg[sl] = jnp.where(b1, colg, FILL) * 0.5 + jnp.where(b2, colg, FILL) * 0.5
        gg[sl] = jnp.where(b1, colb, FILL) * 0.5 + jnp.where(b2, colb, FILL) * 0.5
        return 0

    lax.fori_loop(0, PPW // 16, fin, 0)
    pltpu.sync_copy(zg, out_h.at[0, pl.ds(base, PPW)])
    pltpu.sync_copy(mg, out_h.at[1, pl.ds(base, PPW)])
    pltpu.sync_copy(rg, out_h.at[2, pl.ds(base, PPW)])
    pltpu.sync_copy(gg, out_h.at[3, pl.ds(base, PPW)])


@functools.partial(jax.jit, static_argnames=())
def _run(maskf, rf, gf, bf, bx, by, bz, par, glob):
    prep = pl.kernel(
        _prep_body,
        mesh=_MESH,
        compiler_params=_CPARAMS,
        out_type=(
            jax.ShapeDtypeStruct((C, N), jnp.int32),
            jax.ShapeDtypeStruct((C, N), jnp.float32),
        ),
        scratch_types=[
            pltpu.VMEM((PPW,), jnp.float32),
            pltpu.VMEM((PPW,), jnp.float32),
            pltpu.VMEM((PPW,), jnp.float32),
            pltpu.VMEM((PPW,), jnp.int32),
            pltpu.VMEM((PPW,), jnp.float32),
            pltpu.VMEM((C * NPAR * 16,), jnp.float32),
            pltpu.VMEM((5 * 16,), jnp.float32),
            pltpu.SemaphoreType.DMA,
        ],
    )
    pid, dist = prep(bx, by, bz, par, glob)

    zbuf = pl.kernel(
        _zbuf_body,
        mesh=_MESH,
        compiler_params=_CPARAMS,
        out_type=jax.ShapeDtypeStruct((C, HW), jnp.float32),
        scratch_types=[
            pltpu.VMEM((16384,), jnp.int32),
            pltpu.VMEM((16384,), jnp.float32),
            pltpu.VMEM((SLICE_PIX,), jnp.float32),
            pltpu.SemaphoreType.DMA,
        ],
    )(pid, dist)


    out4 = pl.kernel(
        _sample_body,
        mesh=_MESH,
        compiler_params=_CPARAMS,
        out_type=jax.ShapeDtypeStruct((4, N), jnp.float32),
        scratch_types=(
            [pltpu.VMEM((PPW,), jnp.int32)] * 2
            + [pltpu.VMEM((PPW,), jnp.float32)] * 11
            + [pltpu.SemaphoreType.DMA, pltpu.SemaphoreType.DMA]
        ),
    )(pid, dist, zbuf.reshape(C * HW), maskf, rf, gf, bf)
    return out4


def kernel(mask, rgb, center, angle, K, E):
    lin = jnp.linspace(-0.5, 0.5, GRID, dtype=jnp.float32)
    gx, gy, gz = jnp.meshgrid(lin, lin, lin, indexing="ij")
    bx = gx.reshape(-1)
    by = gy.reshape(-1)
    bz = gz.reshape(-1)

    th = jnp.asarray(angle, jnp.float32)
    ct, st = jnp.cos(th), jnp.sin(th)
    Rt = jnp.transpose(E[:, :3, :3], (0, 2, 1))
    cam_pos = -jnp.einsum("cij,cj->ci", Rt, E[:, :3, 3])

    par = jnp.concatenate(
        [
            E[:, :3, :].reshape(C, 12),
            K.reshape(C, 9),
            cam_pos.reshape(C, 3),
        ],
        axis=1,
    )
    par = jnp.broadcast_to(par[:, :, None], (C, NPAR, 16)).reshape(-1)
    glob = jnp.concatenate(
        [ct[None], st[None], jnp.asarray(center, jnp.float32)]
    )
    glob = jnp.broadcast_to(glob[:, None], (5, 16)).reshape(-1)

    out4 = _run(mask.reshape(C * HW), rgb[:, 0].reshape(-1),
                rgb[:, 1].reshape(-1), rgb[:, 2].reshape(-1),
                bx, by, bz, par, glob)
    return out4.reshape(4, GRID, GRID, GRID)


# speedup vs baseline: 4.2301x; 1.0482x over previous
import functools

import jax
import jax.numpy as jnp
from jax import lax
from jax.experimental import pallas as pl
from jax.experimental.pallas import tpu as pltpu
from jax.experimental.pallas import tpu_sc as plsc

GRID = 64
N = GRID ** 3
C = 8
H = W = 512
HW = H * W
NW = 32
PPW = N // NW
SLICES = 4
SLICE_PIX = HW // SLICES
FILL = 0.45
NONVIS_W = 0.25
HALF = (GRID ** 3) // 32 // 2
MAGIC = 12582912.0
F32_INF = float("inf")

_CPARAMS = pltpu.CompilerParams(needs_layout_passes=False)
_MESH = plsc.VectorSubcoreMesh(core_axis_name="c", subcore_axis_name="s")


def _wid():
    return lax.axis_index("c") * 16 + lax.axis_index("s")


def _any(v_bool):
    return jnp.max(v_bool.astype(jnp.int32), axis=0) != 0


NPAR = 24


def _prep_body(bx_h, by_h, bz_h, par_h, glob_h, pid_h, dist_h,
               bxv, byv, bzv, pidv, dstv, parv, globv, sem):
    w = _wid()
    base = w * PPW
    pltpu.sync_copy(bx_h.at[pl.ds(base, PPW)], bxv)
    pltpu.sync_copy(by_h.at[pl.ds(base, PPW)], byv)
    pltpu.sync_copy(bz_h.at[pl.ds(base, PPW)], bzv)
    pltpu.sync_copy(par_h, parv)
    pltpu.sync_copy(glob_h, globv)

    ct = globv[pl.ds(0, 16)]
    st = globv[pl.ds(16, 16)]
    cx = globv[pl.ds(32, 16)]
    cy = globv[pl.ds(48, 16)]
    cz = globv[pl.ds(64, 16)]

    for c in range(C):
        def P(j, c=c):
            return parv[pl.ds((c * NPAR + j) * 16, 16)]

        e00 = P(0)
        e01 = P(1)
        e02 = P(2)
        e03 = P(3)
        e10 = P(4)
        e11 = P(5)
        e12 = P(6)
        e13 = P(7)
        e20 = P(8)
        e21 = P(9)
        e22 = P(10)
        e23 = P(11)
        k00 = P(12)
        k01 = P(13)
        k02 = P(14)
        k10 = P(15)
        k11 = P(16)
        k12 = P(17)
        k20 = P(18)
        k21 = P(19)
        k22 = P(20)
        px0 = P(21)
        py0 = P(22)
        pz0 = P(23)

        def body(i, _, c=c):
            sl = pl.ds(i * 16, 16)
            bxx = bxv[sl]
            byy = byv[sl]
            bzz = bzv[sl]
            x = ct * bxx - st * byy + cx
            y = st * bxx + ct * byy + cy
            z = bzz + cz
            camx = e00 * x + e01 * y + e02 * z + e03
            camy = e10 * x + e11 * y + e12 * z + e13
            camz = e20 * x + e21 * y + e22 * z + e23
            pixx = k00 * camx + k01 * camy + k02 * camz
            pixy = k10 * camx + k11 * camy + k12 * camz
            pixz = k20 * camx + k21 * camy + k22 * camz
            zc = pixz + 1e-8
            u = pixx / zc
            v = pixy / zc
            ru = (u + MAGIC) - MAGIC
            rv = (v + MAGIC) - MAGIC
            ru = jnp.minimum(jnp.maximum(ru, 0.0), 511.0)
            rv = jnp.minimum(jnp.maximum(rv, 0.0), 511.0)
            ipx = ru.astype(jnp.int32)
            ipy = rv.astype(jnp.int32)
            pidv[sl] = ipy * W + ipx
            dx = x - px0
            dy = y - py0
            dz = z - pz0
            dstv[sl] = dx * dx + dy * dy + dz * dz
            return 0

        lax.fori_loop(0, PPW // 16, body, 0)
        pltpu.sync_copy(pidv, pid_h.at[c, pl.ds(base, PPW)])
        pltpu.sync_copy(dstv, dist_h.at[c, pl.ds(base, PPW)])


def _zbuf_body(pid_h, dist_h, zbuf_h, pidc, dstc, zs, sem):
    w = _wid()
    cam = w // SLICES
    sl_i = w % SLICES
    pid_base = sl_i * SLICE_PIX

    def init(i, _):
        for u in range(8):
            zs[pl.ds(i * 128 + u * 16, 16)] = jnp.full((16,), F32_INF, jnp.float32)
        return 0

    lax.fori_loop(0, SLICE_PIX // 128, init, 0)

    CH = 16384
    for chunk in range(N // CH):
        pltpu.sync_copy(pid_h.at[cam, pl.ds(chunk * CH, CH)], pidc)
        pltpu.sync_copy(dist_h.at[cam, pl.ds(chunk * CH, CH)], dstc)

        UNROLL = 8

        def body(i, _):
            sorted_vs = []
            for u in range(UNROLL):
                sl = pl.ds(i * (16 * UNROLL) + u * 16, 16)
                pv = pidc[sl]
                dv = dstc[sl]
                li = pv - pid_base
                m = (li >= 0) & (li < SLICE_PIX)
                dvm = jnp.where(m, dv, F32_INF)
                lic = jnp.minimum(jnp.maximum(li, 0), SLICE_PIX - 1)
                sorted_vs.append(plsc.sort_key_val(dvm, lic, descending=True))
            for ds_, ls_ in sorted_vs:
                cur = plsc.load_gather(zs, [ls_])
                wm = ds_ < cur
                plsc.store_scatter(zs, [ls_], ds_, mask=wm)
            return 0

        lax.fori_loop(0, CH // (16 * UNROLL), body, 0)

    pltpu.sync_copy(zs, zbuf_h.at[cam, pl.ds(pid_base, SLICE_PIX)])


def _sample_body(pid_h, dist_h, zbuf_h, mask_h, r_h, g_h, b_h, out_h,
                 pidb, idxz, dstb, zg, mg, rg, gg, bg,
                 msum, wsum, cr, cg, cb, sem, sem2):
    w = _wid()
    base = w * PPW

    def zero(i, _):
        sl = pl.ds(i * 16, 16)
        zv = jnp.zeros((16,), jnp.float32)
        msum[sl] = zv
        wsum[sl] = zv
        cr[sl] = zv
        cg[sl] = zv
        cb[sl] = zv
        return 0

    lax.fori_loop(0, PPW // 16, zero, 0)

    for c in range(C):
        pltpu.sync_copy(pid_h.at[c, pl.ds(base, PPW)], pidb)
        pltpu.sync_copy(dist_h.at[c, pl.ds(base, PPW)], dstb)

        def off(i, _, c=c):
            sl = pl.ds(i * 16, 16)
            idxz[sl] = pidb[sl] + (c * HW)
            return 0

        lax.fori_loop(0, PPW // 16, off, 0)

        d1 = pltpu.async_copy(zbuf_h.at[idxz], zg, sem)
        d2 = pltpu.async_copy(mask_h.at[idxz], mg, sem2)
        d3 = pltpu.async_copy(r_h.at[idxz], rg, sem)
        d4 = pltpu.async_copy(g_h.at[idxz], gg, sem2)
        d5 = pltpu.async_copy(b_h.at[idxz], bg, sem)
        d1.wait()
        d2.wait()
        d3.wait()
        d4.wait()
        d5.wait()

        def acc(i, _):
            sl = pl.ds(i * 16, 16)
            vis = dstb[sl] <= zg[sl]
            wv = jnp.where(vis, 1.0, NONVIS_W).astype(jnp.float32)
            msum[sl] = msum[sl] + mg[sl]
            wsum[sl] = wsum[sl] + wv
            cr[sl] = cr[sl] + wv * rg[sl]
            cg[sl] = cg[sl] + wv * gg[sl]
            cb[sl] = cb[sl] + wv * bg[sl]
            return 0

        lax.fori_loop(0, PPW // 16, acc, 0)

    def fin(i, _):
        sl = pl.ds(i * 16, 16)
        mv = msum[sl] * (1.0 / C)
        b1 = mv >= 1.0
        b2 = mv >= (C - 1) / C
        den = jnp.maximum(wsum[sl], 1e-8)
        colr = cr[sl] / den
        colg = cg[sl] / den
        colb = cb[sl] / den
        zg[sl] = b1.astype(jnp.float32) * 0.5 + b2.astype(jnp.float32) * 0.5
        mg[sl] = jnp.where(b1, colr, FILL) * 0.5 + jnp.where(b2, colr, FILL) * 0.5
        rg[sl] = jnp.where(b1, colg, FILL) * 0.5 + jnp.where(b2, colg, FILL) * 0.5
        gg[sl] = jnp.where(b1, colb, FILL) * 0.5 + jnp.where(b2, colb, FILL) * 0.5
        return 0

    lax.fori_loop(0, PPW // 16, fin, 0)
    pltpu.sync_copy(zg, out_h.at[0, pl.ds(base, PPW)])
    pltpu.sync_copy(mg, out_h.at[1, pl.ds(base, PPW)])
    pltpu.sync_copy(rg, out_h.at[2, pl.ds(base, PPW)])
    pltpu.sync_copy(gg, out_h.at[3, pl.ds(base, PPW)])


@functools.partial(jax.jit, static_argnames=())
def _run(maskf, rf, gf, bf, bx, by, bz, par, glob):
    prep = pl.kernel(
        _prep_body,
        mesh=_MESH,
        compiler_params=_CPARAMS,
        out_type=(
            jax.ShapeDtypeStruct((C, N), jnp.int32),
            jax.ShapeDtypeStruct((C, N), jnp.float32),
        ),
        scratch_types=[
            pltpu.VMEM((PPW,), jnp.float32),
            pltpu.VMEM((PPW,), jnp.float32),
            pltpu.VMEM((PPW,), jnp.float32),
            pltpu.VMEM((PPW,), jnp.int32),
            pltpu.VMEM((PPW,), jnp.float32),
            pltpu.VMEM((C * NPAR * 16,), jnp.float32),
            pltpu.VMEM((5 * 16,), jnp.float32),
            pltpu.SemaphoreType.DMA,
        ],
    )
    pid, dist = prep(bx, by, bz, par, glob)

    zbuf = pl.kernel(
        _zbuf_body,
        mesh=_MESH,
        compiler_params=_CPARAMS,
        out_type=jax.ShapeDtypeStruct((C, HW), jnp.float32),
        scratch_types=[
            pltpu.VMEM((16384,), jnp.int32),
            pltpu.VMEM((16384,), jnp.float32),
            pltpu.VMEM((SLICE_PIX,), jnp.float32),
            pltpu.SemaphoreType.DMA,
        ],
    )(pid, dist)


    out4 = pl.kernel(
        _sample_body,
        mesh=_MESH,
        compiler_params=_CPARAMS,
        out_type=jax.ShapeDtypeStruct((4, N), jnp.float32),
        scratch_types=(
            [pltpu.VMEM((PPW,), jnp.int32)] * 2
            + [pltpu.VMEM((PPW,), jnp.float32)] * 11
            + [pltpu.SemaphoreType.DMA, pltpu.SemaphoreType.DMA]
        ),
    )(pid, dist, zbuf.reshape(C * HW), maskf, rf, gf, bf)
    return out4


def kernel(mask, rgb, center, angle, K, E):
    lin = jnp.linspace(-0.5, 0.5, GRID, dtype=jnp.float32)
    gx, gy, gz = jnp.meshgrid(lin, lin, lin, indexing="ij")
    bx = gx.reshape(-1)
    by = gy.reshape(-1)
    bz = gz.reshape(-1)

    th = jnp.asarray(angle, jnp.float32)
    ct, st = jnp.cos(th), jnp.sin(th)
    Rt = jnp.transpose(E[:, :3, :3], (0, 2, 1))
    cam_pos = -jnp.einsum("cij,cj->ci", Rt, E[:, :3, 3])

    par = jnp.concatenate(
        [
            E[:, :3, :].reshape(C, 12),
            K.reshape(C, 9),
            cam_pos.reshape(C, 3),
        ],
        axis=1,
    )
    par = jnp.broadcast_to(par[:, :, None], (C, NPAR, 16)).reshape(-1)
    glob = jnp.concatenate(
        [ct[None], st[None], jnp.asarray(center, jnp.float32)]
    )
    glob = jnp.broadcast_to(glob[:, None], (5, 16)).reshape(-1)

    out4 = _run(mask.reshape(C * HW), rgb[:, 0].reshape(-1),
                rgb[:, 1].reshape(-1), rgb[:, 2].reshape(-1),
                bx, by, bz, par, glob)
    return out4.reshape(4, GRID, GRID, GRID)


# speedup vs baseline: 4.2681x; 1.0090x over previous
import functools

import jax
import jax.numpy as jnp
from jax import lax
from jax.experimental import pallas as pl
from jax.experimental.pallas import tpu as pltpu
from jax.experimental.pallas import tpu_sc as plsc

GRID = 64
N = GRID ** 3
C = 8
H = W = 512
HW = H * W
NW = 32
PPW = N // NW
SLICES = 4
SLICE_PIX = HW // SLICES
FILL = 0.45
NONVIS_W = 0.25
HALF = (GRID ** 3) // 32 // 2
MAGIC = 12582912.0
F32_INF = float("inf")

_CPARAMS = pltpu.CompilerParams(needs_layout_passes=False)
_MESH = plsc.VectorSubcoreMesh(core_axis_name="c", subcore_axis_name="s")


def _wid():
    return lax.axis_index("c") * 16 + lax.axis_index("s")


def _any(v_bool):
    return jnp.max(v_bool.astype(jnp.int32), axis=0) != 0


NPAR = 24


def _prep_body(bx_h, by_h, bz_h, par_h, glob_h, pid_h, dist_h,
               bxv, byv, bzv, pidv, dstv, parv, globv, sem):
    w = _wid()
    base = w * PPW
    pltpu.sync_copy(bx_h.at[pl.ds(base, PPW)], bxv)
    pltpu.sync_copy(by_h.at[pl.ds(base, PPW)], byv)
    pltpu.sync_copy(bz_h.at[pl.ds(base, PPW)], bzv)
    pltpu.sync_copy(par_h, parv)
    pltpu.sync_copy(glob_h, globv)

    ct = globv[pl.ds(0, 16)]
    st = globv[pl.ds(16, 16)]
    cx = globv[pl.ds(32, 16)]
    cy = globv[pl.ds(48, 16)]
    cz = globv[pl.ds(64, 16)]

    for c in range(C):
        def P(j, c=c):
            return parv[pl.ds((c * NPAR + j) * 16, 16)]

        e00 = P(0)
        e01 = P(1)
        e02 = P(2)
        e03 = P(3)
        e10 = P(4)
        e11 = P(5)
        e12 = P(6)
        e13 = P(7)
        e20 = P(8)
        e21 = P(9)
        e22 = P(10)
        e23 = P(11)
        k00 = P(12)
        k01 = P(13)
        k02 = P(14)
        k10 = P(15)
        k11 = P(16)
        k12 = P(17)
        k20 = P(18)
        k21 = P(19)
        k22 = P(20)
        px0 = P(21)
        py0 = P(22)
        pz0 = P(23)

        def body(i, _, c=c):
          for u in range(2):
            sl = pl.ds(i * 32 + u * 16, 16)
            bxx = bxv[sl]
            byy = byv[sl]
            bzz = bzv[sl]
            x = ct * bxx - st * byy + cx
            y = st * bxx + ct * byy + cy
            z = bzz + cz
            camx = e00 * x + e01 * y + e02 * z + e03
            camy = e10 * x + e11 * y + e12 * z + e13
            camz = e20 * x + e21 * y + e22 * z + e23
            pixx = k00 * camx + k01 * camy + k02 * camz
            pixy = k10 * camx + k11 * camy + k12 * camz
            pixz = k20 * camx + k21 * camy + k22 * camz
            zc = pixz + 1e-8
            u = pixx / zc
            v = pixy / zc
            ru = (u + MAGIC) - MAGIC
            rv = (v + MAGIC) - MAGIC
            ru = jnp.minimum(jnp.maximum(ru, 0.0), 511.0)
            rv = jnp.minimum(jnp.maximum(rv, 0.0), 511.0)
            ipx = ru.astype(jnp.int32)
            ipy = rv.astype(jnp.int32)
            pidv[sl] = ipy * W + ipx
            dx = x - px0
            dy = y - py0
            dz = z - pz0
            dstv[sl] = dx * dx + dy * dy + dz * dz
          return 0

        lax.fori_loop(0, PPW // 32, body, 0)
        pltpu.sync_copy(pidv, pid_h.at[c, pl.ds(base, PPW)])
        pltpu.sync_copy(dstv, dist_h.at[c, pl.ds(base, PPW)])


def _zbuf_body(pid_h, dist_h, zbuf_h, pidc, dstc, zs, sem):
    w = _wid()
    cam = w // SLICES
    sl_i = w % SLICES
    pid_base = sl_i * SLICE_PIX

    def init(i, _):
        for u in range(8):
            zs[pl.ds(i * 128 + u * 16, 16)] = jnp.full((16,), F32_INF, jnp.float32)
        return 0

    lax.fori_loop(0, SLICE_PIX // 128, init, 0)

    CH = 16384
    for chunk in range(N // CH):
        pltpu.sync_copy(pid_h.at[cam, pl.ds(chunk * CH, CH)], pidc)
        pltpu.sync_copy(dist_h.at[cam, pl.ds(chunk * CH, CH)], dstc)

        UNROLL = 8

        def body(i, _):
            sorted_vs = []
            for u in range(UNROLL):
                sl = pl.ds(i * (16 * UNROLL) + u * 16, 16)
                pv = pidc[sl]
                dv = dstc[sl]
                li = pv - pid_base
                m = (li >= 0) & (li < SLICE_PIX)
                dvm = jnp.where(m, dv, F32_INF)
                lic = jnp.minimum(jnp.maximum(li, 0), SLICE_PIX - 1)
                sorted_vs.append(plsc.sort_key_val(dvm, lic, descending=True))
            for ds_, ls_ in sorted_vs:
                cur = plsc.load_gather(zs, [ls_])
                wm = ds_ < cur
                plsc.store_scatter(zs, [ls_], ds_, mask=wm)
            return 0

        lax.fori_loop(0, CH // (16 * UNROLL), body, 0)

    pltpu.sync_copy(zs, zbuf_h.at[cam, pl.ds(pid_base, SLICE_PIX)])


def _sample_body(pid_h, dist_h, zbuf_h, mask_h, r_h, g_h, b_h, out_h,
                 pidb, idxz, dstb, zg, mg, rg, gg, bg,
                 msum, wsum, cr, cg, cb, sem, sem2):
    w = _wid()
    base = w * PPW

    def zero(i, _):
        sl = pl.ds(i * 16, 16)
        zv = jnp.zeros((16,), jnp.float32)
        msum[sl] = zv
        wsum[sl] = zv
        cr[sl] = zv
        cg[sl] = zv
        cb[sl] = zv
        return 0

    lax.fori_loop(0, PPW // 16, zero, 0)

    for c in range(C):
        pltpu.sync_copy(pid_h.at[c, pl.ds(base, PPW)], pidb)
        pltpu.sync_copy(dist_h.at[c, pl.ds(base, PPW)], dstb)

        def off(i, _, c=c):
            for u in range(4):
                sl = pl.ds(i * 64 + u * 16, 16)
                idxz[sl] = pidb[sl] + (c * HW)
            return 0

        lax.fori_loop(0, PPW // 64, off, 0)

        d1 = pltpu.async_copy(zbuf_h.at[idxz], zg, sem)
        d2 = pltpu.async_copy(mask_h.at[idxz], mg, sem2)
        d3 = pltpu.async_copy(r_h.at[idxz], rg, sem)
        d4 = pltpu.async_copy(g_h.at[idxz], gg, sem2)
        d5 = pltpu.async_copy(b_h.at[idxz], bg, sem)
        d1.wait()
        d2.wait()
        d3.wait()
        d4.wait()
        d5.wait()

        def acc(i, _):
            for u in range(4):
                sl = pl.ds(i * 64 + u * 16, 16)
                vis = dstb[sl] <= zg[sl]
                wv = jnp.where(vis, 1.0, NONVIS_W).astype(jnp.float32)
                msum[sl] = msum[sl] + mg[sl]
                wsum[sl] = wsum[sl] + wv
                cr[sl] = cr[sl] + wv * rg[sl]
                cg[sl] = cg[sl] + wv * gg[sl]
                cb[sl] = cb[sl] + wv * bg[sl]
            return 0

        lax.fori_loop(0, PPW // 64, acc, 0)

    def fin(i, _):
        sl = pl.ds(i * 16, 16)
        mv = msum[sl] * (1.0 / C)
        b1 = mv >= 1.0
        b2 = mv >= (C - 1) / C
        den = jnp.maximum(wsum[sl], 1e-8)
        colr = cr[sl] / den
        colg = cg[sl] / den
        colb = cb[sl] / den
        zg[sl] = b1.astype(jnp.float32) * 0.5 + b2.astype(jnp.float32) * 0.5
        mg[sl] = jnp.where(b1, colr, FILL) * 0.5 + jnp.where(b2, colr, FILL) * 0.5
        rg[sl] = jnp.where(b1, colg, FILL) * 0.5 + jnp.where(b2, colg, FILL) * 0.5
        gg[sl] = jnp.where(b1, colb, FILL) * 0.5 + jnp.where(b2, colb, FILL) * 0.5
        return 0

    lax.fori_loop(0, PPW // 16, fin, 0)
    pltpu.sync_copy(zg, out_h.at[0, pl.ds(base, PPW)])
    pltpu.sync_copy(mg, out_h.at[1, pl.ds(base, PPW)])
    pltpu.sync_copy(rg, out_h.at[2, pl.ds(base, PPW)])
    pltpu.sync_copy(gg, out_h.at[3, pl.ds(base, PPW)])


@functools.partial(jax.jit, static_argnames=())
def _run(maskf, rf, gf, bf, bx, by, bz, par, glob):
    prep = pl.kernel(
        _prep_body,
        mesh=_MESH,
        compiler_params=_CPARAMS,
        out_type=(
            jax.ShapeDtypeStruct((C, N), jnp.int32),
            jax.ShapeDtypeStruct((C, N), jnp.float32),
        ),
        scratch_types=[
            pltpu.VMEM((PPW,), jnp.float32),
            pltpu.VMEM((PPW,), jnp.float32),
            pltpu.VMEM((PPW,), jnp.float32),
            pltpu.VMEM((PPW,), jnp.int32),
            pltpu.VMEM((PPW,), jnp.float32),
            pltpu.VMEM((C * NPAR * 16,), jnp.float32),
            pltpu.VMEM((5 * 16,), jnp.float32),
            pltpu.SemaphoreType.DMA,
        ],
    )
    pid, dist = prep(bx, by, bz, par, glob)

    zbuf = pl.kernel(
        _zbuf_body,
        mesh=_MESH,
        compiler_params=_CPARAMS,
        out_type=jax.ShapeDtypeStruct((C, HW), jnp.float32),
        scratch_types=[
            pltpu.VMEM((16384,), jnp.int32),
            pltpu.VMEM((16384,), jnp.float32),
            pltpu.VMEM((SLICE_PIX,), jnp.float32),
            pltpu.SemaphoreType.DMA,
        ],
    )(pid, dist)


    out4 = pl.kernel(
        _sample_body,
        mesh=_MESH,
        compiler_params=_CPARAMS,
        out_type=jax.ShapeDtypeStruct((4, N), jnp.float32),
        scratch_types=(
            [pltpu.VMEM((PPW,), jnp.int32)] * 2
            + [pltpu.VMEM((PPW,), jnp.float32)] * 11
            + [pltpu.SemaphoreType.DMA, pltpu.SemaphoreType.DMA]
        ),
    )(pid, dist, zbuf.reshape(C * HW), maskf, rf, gf, bf)
    return out4


def kernel(mask, rgb, center, angle, K, E):
    lin = jnp.linspace(-0.5, 0.5, GRID, dtype=jnp.float32)
    gx, gy, gz = jnp.meshgrid(lin, lin, lin, indexing="ij")
    bx = gx.reshape(-1)
    by = gy.reshape(-1)
    bz = gz.reshape(-1)

    th = jnp.asarray(angle, jnp.float32)
    ct, st = jnp.cos(th), jnp.sin(th)
    Rt = jnp.transpose(E[:, :3, :3], (0, 2, 1))
    cam_pos = -jnp.einsum("cij,cj->ci", Rt, E[:, :3, 3])

    par = jnp.concatenate(
        [
            E[:, :3, :].reshape(C, 12),
            K.reshape(C, 9),
            cam_pos.reshape(C, 3),
        ],
        axis=1,
    )
    par = jnp.broadcast_to(par[:, :, None], (C, NPAR, 16)).reshape(-1)
    glob = jnp.concatenate(
        [ct[None], st[None], jnp.asarray(center, jnp.float32)]
    )
    glob = jnp.broadcast_to(glob[:, None], (5, 16)).reshape(-1)

    out4 = _run(mask.reshape(C * HW), rgb[:, 0].reshape(-1),
                rgb[:, 1].reshape(-1), rgb[:, 2].reshape(-1),
                bx, by, bz, par, glob)
    return out4.reshape(4, GRID, GRID, GRID)


# speedup vs baseline: 4.2685x; 1.0001x over previous
import functools

import jax
import jax.numpy as jnp
from jax import lax
from jax.experimental import pallas as pl
from jax.experimental.pallas import tpu as pltpu
from jax.experimental.pallas import tpu_sc as plsc

GRID = 64
N = GRID ** 3
C = 8
H = W = 512
HW = H * W
NW = 32
PPW = N // NW
SLICES = 4
SLICE_PIX = HW // SLICES
FILL = 0.45
NONVIS_W = 0.25
MAGIC = 12582912.0
F32_INF = float("inf")

_CPARAMS = pltpu.CompilerParams(needs_layout_passes=False)
_MESH = plsc.VectorSubcoreMesh(core_axis_name="c", subcore_axis_name="s")


def _wid():
    return lax.axis_index("c") * 16 + lax.axis_index("s")


NPAR = 24


def _prep_body(bx_h, by_h, bz_h, par_h, glob_h, pid_h, dist_h,
               bxv, byv, bzv, pidv, dstv, parv, globv, sem):
    w = _wid()
    base = w * PPW
    pltpu.sync_copy(bx_h.at[pl.ds(base, PPW)], bxv)
    pltpu.sync_copy(by_h.at[pl.ds(base, PPW)], byv)
    pltpu.sync_copy(bz_h.at[pl.ds(base, PPW)], bzv)
    pltpu.sync_copy(par_h, parv)
    pltpu.sync_copy(glob_h, globv)

    ct = globv[pl.ds(0, 16)]
    st = globv[pl.ds(16, 16)]
    cx = globv[pl.ds(32, 16)]
    cy = globv[pl.ds(48, 16)]
    cz = globv[pl.ds(64, 16)]

    for c in range(C):
        def P(j, c=c):
            return parv[pl.ds((c * NPAR + j) * 16, 16)]

        e00 = P(0)
        e01 = P(1)
        e02 = P(2)
        e03 = P(3)
        e10 = P(4)
        e11 = P(5)
        e12 = P(6)
        e13 = P(7)
        e20 = P(8)
        e21 = P(9)
        e22 = P(10)
        e23 = P(11)
        k00 = P(12)
        k01 = P(13)
        k02 = P(14)
        k10 = P(15)
        k11 = P(16)
        k12 = P(17)
        k20 = P(18)
        k21 = P(19)
        k22 = P(20)
        px0 = P(21)
        py0 = P(22)
        pz0 = P(23)

        def body(i, _, c=c):
          for u in range(2):
            sl = pl.ds(i * 32 + u * 16, 16)
            bxx = bxv[sl]
            byy = byv[sl]
            bzz = bzv[sl]
            x = ct * bxx - st * byy + cx
            y = st * bxx + ct * byy + cy
            z = bzz + cz
            camx = e00 * x + e01 * y + e02 * z + e03
            camy = e10 * x + e11 * y + e12 * z + e13
            camz = e20 * x + e21 * y + e22 * z + e23
            pixx = k00 * camx + k01 * camy + k02 * camz
            pixy = k10 * camx + k11 * camy + k12 * camz
            pixz = k20 * camx + k21 * camy + k22 * camz
            zc = pixz + 1e-8
            u = pixx / zc
            v = pixy / zc
            ru = (u + MAGIC) - MAGIC
            rv = (v + MAGIC) - MAGIC
            ru = jnp.minimum(jnp.maximum(ru, 0.0), 511.0)
            rv = jnp.minimum(jnp.maximum(rv, 0.0), 511.0)
            ipx = ru.astype(jnp.int32)
            ipy = rv.astype(jnp.int32)
            pidv[sl] = ipy * W + ipx
            dx = x - px0
            dy = y - py0
            dz = z - pz0
            dstv[sl] = dx * dx + dy * dy + dz * dz
          return 0

        lax.fori_loop(0, PPW // 32, body, 0)
        pltpu.sync_copy(pidv, pid_h.at[c, pl.ds(base, PPW)])
        pltpu.sync_copy(dstv, dist_h.at[c, pl.ds(base, PPW)])


def _zbuf_body(pid_h, dist_h, zbuf_h, pidc, dstc, zs, sem):
    w = _wid()
    cam = w // SLICES
    sl_i = w % SLICES
    pid_base = sl_i * SLICE_PIX

    def init(i, _):
        for u in range(8):
            zs[pl.ds(i * 128 + u * 16, 16)] = jnp.full((16,), F32_INF, jnp.float32)
        return 0

    lax.fori_loop(0, SLICE_PIX // 128, init, 0)

    CH = 16384
    for chunk in range(N // CH):
        pltpu.sync_copy(pid_h.at[cam, pl.ds(chunk * CH, CH)], pidc)
        pltpu.sync_copy(dist_h.at[cam, pl.ds(chunk * CH, CH)], dstc)

        UNROLL = 8

        def body(i, _):
            sorted_vs = []
            for u in range(UNROLL):
                sl = pl.ds(i * (16 * UNROLL) + u * 16, 16)
                pv = pidc[sl]
                dv = dstc[sl]
                li = pv - pid_base
                m = (li >= 0) & (li < SLICE_PIX)
                dvm = jnp.where(m, dv, F32_INF)
                lic = jnp.minimum(jnp.maximum(li, 0), SLICE_PIX - 1)
                sorted_vs.append(plsc.sort_key_val(dvm, lic, descending=True))
            for ds_, ls_ in sorted_vs:
                cur = plsc.load_gather(zs, [ls_])
                wm = ds_ < cur
                plsc.store_scatter(zs, [ls_], ds_, mask=wm)
            return 0

        lax.fori_loop(0, CH // (16 * UNROLL), body, 0)

    pltpu.sync_copy(zs, zbuf_h.at[cam, pl.ds(pid_base, SLICE_PIX)])


def _sample_body(pid_h, dist_h, zbuf_h, mask_h, r_h, g_h, b_h, out_h,
                 pidb, idxz, dstb, zg, mg, rg, gg, bg,
                 msum, wsum, cr, cg, cb, sem, sem2):
    w = _wid()
    base = w * PPW

    def zero(i, _):
        sl = pl.ds(i * 16, 16)
        zv = jnp.zeros((16,), jnp.float32)
        msum[sl] = zv
        wsum[sl] = zv
        cr[sl] = zv
        cg[sl] = zv
        cb[sl] = zv
        return 0

    lax.fori_loop(0, PPW // 16, zero, 0)

    for c in range(C):
        pltpu.sync_copy(pid_h.at[c, pl.ds(base, PPW)], pidb)
        pltpu.sync_copy(dist_h.at[c, pl.ds(base, PPW)], dstb)

        def off(i, _, c=c):
            for u in range(4):
                sl = pl.ds(i * 64 + u * 16, 16)
                idxz[sl] = pidb[sl] + (c * HW)
            return 0

        lax.fori_loop(0, PPW // 64, off, 0)

        d1 = pltpu.async_copy(zbuf_h.at[idxz], zg, sem)
        d2 = pltpu.async_copy(mask_h.at[idxz], mg, sem2)
        d3 = pltpu.async_copy(r_h.at[idxz], rg, sem)
        d4 = pltpu.async_copy(g_h.at[idxz], gg, sem2)
        d5 = pltpu.async_copy(b_h.at[idxz], bg, sem)
        d1.wait()
        d2.wait()
        d3.wait()
        d4.wait()
        d5.wait()

        def acc(i, _):
            for u in range(4):
                sl = pl.ds(i * 64 + u * 16, 16)
                vis = dstb[sl] <= zg[sl]
                wv = jnp.where(vis, 1.0, NONVIS_W).astype(jnp.float32)
                msum[sl] = msum[sl] + mg[sl]
                wsum[sl] = wsum[sl] + wv
                cr[sl] = cr[sl] + wv * rg[sl]
                cg[sl] = cg[sl] + wv * gg[sl]
                cb[sl] = cb[sl] + wv * bg[sl]
            return 0

        lax.fori_loop(0, PPW // 64, acc, 0)

    def fin(i, _):
        sl = pl.ds(i * 16, 16)
        mv = msum[sl] * (1.0 / C)
        b1 = mv >= 1.0
        b2 = mv >= (C - 1) / C
        den = jnp.maximum(wsum[sl], 1e-8)
        colr = cr[sl] / den
        colg = cg[sl] / den
        colb = cb[sl] / den
        zg[sl] = b1.astype(jnp.float32) * 0.5 + b2.astype(jnp.float32) * 0.5
        mg[sl] = jnp.where(b1, colr, FILL) * 0.5 + jnp.where(b2, colr, FILL) * 0.5
        rg[sl] = jnp.where(b1, colg, FILL) * 0.5 + jnp.where(b2, colg, FILL) * 0.5
        gg[sl] = jnp.where(b1, colb, FILL) * 0.5 + jnp.where(b2, colb, FILL) * 0.5
        return 0

    lax.fori_loop(0, PPW // 16, fin, 0)
    pltpu.sync_copy(zg, out_h.at[0, pl.ds(base, PPW)])
    pltpu.sync_copy(mg, out_h.at[1, pl.ds(base, PPW)])
    pltpu.sync_copy(rg, out_h.at[2, pl.ds(base, PPW)])
    pltpu.sync_copy(gg, out_h.at[3, pl.ds(base, PPW)])


@functools.partial(jax.jit, static_argnames=())
def _run(maskf, rf, gf, bf, bx, by, bz, par, glob):
    prep = pl.kernel(
        _prep_body,
        mesh=_MESH,
        compiler_params=_CPARAMS,
        out_type=(
            jax.ShapeDtypeStruct((C, N), jnp.int32),
            jax.ShapeDtypeStruct((C, N), jnp.float32),
        ),
        scratch_types=[
            pltpu.VMEM((PPW,), jnp.float32),
            pltpu.VMEM((PPW,), jnp.float32),
            pltpu.VMEM((PPW,), jnp.float32),
            pltpu.VMEM((PPW,), jnp.int32),
            pltpu.VMEM((PPW,), jnp.float32),
            pltpu.VMEM((C * NPAR * 16,), jnp.float32),
            pltpu.VMEM((5 * 16,), jnp.float32),
            pltpu.SemaphoreType.DMA,
        ],
    )
    pid, dist = prep(bx, by, bz, par, glob)

    zbuf = pl.kernel(
        _zbuf_body,
        mesh=_MESH,
        compiler_params=_CPARAMS,
        out_type=jax.ShapeDtypeStruct((C, HW), jnp.float32),
        scratch_types=[
            pltpu.VMEM((16384,), jnp.int32),
            pltpu.VMEM((16384,), jnp.float32),
            pltpu.VMEM((SLICE_PIX,), jnp.float32),
            pltpu.SemaphoreType.DMA,
        ],
    )(pid, dist)


    out4 = pl.kernel(
        _sample_body,
        mesh=_MESH,
        compiler_params=_CPARAMS,
        out_type=jax.ShapeDtypeStruct((4, N), jnp.float32),
        scratch_types=(
            [pltpu.VMEM((PPW,), jnp.int32)] * 2
            + [pltpu.VMEM((PPW,), jnp.float32)] * 11
            + [pltpu.SemaphoreType.DMA, pltpu.SemaphoreType.DMA]
        ),
    )(pid, dist, zbuf.reshape(C * HW), maskf, rf, gf, bf)
    return out4


def kernel(mask, rgb, center, angle, K, E):
    lin = jnp.linspace(-0.5, 0.5, GRID, dtype=jnp.float32)
    gx, gy, gz = jnp.meshgrid(lin, lin, lin, indexing="ij")
    bx = gx.reshape(-1)
    by = gy.reshape(-1)
    bz = gz.reshape(-1)

    th = jnp.asarray(angle, jnp.float32)
    ct, st = jnp.cos(th), jnp.sin(th)
    Rt = jnp.transpose(E[:, :3, :3], (0, 2, 1))
    cam_pos = -jnp.einsum("cij,cj->ci", Rt, E[:, :3, 3])

    par = jnp.concatenate(
        [
            E[:, :3, :].reshape(C, 12),
            K.reshape(C, 9),
            cam_pos.reshape(C, 3),
        ],
        axis=1,
    )
    par = jnp.broadcast_to(par[:, :, None], (C, NPAR, 16)).reshape(-1)
    glob = jnp.concatenate(
        [ct[None], st[None], jnp.asarray(center, jnp.float32)]
    )
    glob = jnp.broadcast_to(glob[:, None], (5, 16)).reshape(-1)

    out4 = _run(mask.reshape(C * HW), rgb[:, 0].reshape(-1),
                rgb[:, 1].reshape(-1), rgb[:, 2].reshape(-1),
                bx, by, bz, par, glob)
    return out4.reshape(4, GRID, GRID, GRID)
